# Initial kernel scaffold; baseline (speedup 1.0000x reference)
import jax
import jax.numpy as jnp
from jax.experimental import pallas as pl


def kernel(node_fea, idx1, idx2, edge_fea, params):
    raise NotImplementedError("write your pallas kernel here")


# speedup vs baseline: 3.9964x; 3.9964x over previous
import dataclasses
import functools

import jax
import jax.numpy as jnp
from jax import lax
from jax.experimental import pallas as pl
from jax.experimental.pallas import tpu as pltpu
from jax.experimental.pallas import tpu_sc as plsc

_DIM = 128
_N = 10000
_E = 320000
_EPS = 1e-5

_F32 = jnp.float32
_PF32 = dict(preferred_element_type=jnp.float32)

_NC, _NS = 2, 16
_NW = _NC * _NS
_GW = 128
_EP = 327680
_CH = 128
_NCH = _E // _CH // _NW
_NT = 10240
_STR = _NT // _NS

_BE = 2000


def _leaky(x):
    return jnp.where(x >= 0, x, 0.2 * x)


def _pre_body(node_ref, w1a_ref, w1b_ref, be1_ref, a_ref, b_ref):
    n = node_ref[...]
    a_ref[...] = jnp.dot(n, w1a_ref[...], **_PF32) + be1_ref[0:1, :]
    b_ref[...] = jnp.dot(n, w1b_ref[...], **_PF32)


def _pretransform(node_fea, w1a, w1b, be1):
    return pl.pallas_call(
        _pre_body,
        out_shape=(
            jax.ShapeDtypeStruct((_N, _DIM), _F32),
            jax.ShapeDtypeStruct((_N, _DIM), _F32),
        ),
    )(node_fea, w1a, w1b, be1)


def _sc_gather(table_a, table_b, idx1, idx2):
    mesh = plsc.VectorSubcoreMesh(core_axis_name="c", subcore_axis_name="s")

    @functools.partial(
        pl.kernel,
        out_type=(
            jax.ShapeDtypeStruct((_EP, _DIM), _F32),
            jax.ShapeDtypeStruct((_EP, _DIM), _F32),
        ),
        mesh=mesh,
    )
    def k(ta_hbm, tb_hbm, i1_hbm, i2_hbm, oa_hbm, ob_hbm):
        def body(i1_vmem, i2_vmem, oa_vmem, ob_vmem):
            pltpu.sync_copy(ta_hbm.at[i1_vmem.at[0]], oa_vmem)
            pltpu.sync_copy(tb_hbm.at[i2_vmem.at[0]], ob_vmem)

        pltpu.emit_pipeline(
            body,
            grid=(_EP // _GW,),
            in_specs=[
                pl.BlockSpec((1, _GW), lambda i: (0, i)),
                pl.BlockSpec((1, _GW), lambda i: (0, i)),
            ],
            out_specs=[
                pl.BlockSpec((_GW, _DIM), lambda i: (i, 0)),
                pl.BlockSpec((_GW, _DIM), lambda i: (i, 0)),
            ],
            core_axis_name=("c", "s"),
            dimension_semantics=(pltpu.PARALLEL,),
        )(i1_hbm, i2_hbm, oa_hbm, ob_hbm)

    pad = (jnp.arange(_EP - _E, dtype=jnp.int32) % _N)
    i1p = jnp.concatenate([idx1, pad]).reshape(1, _EP)
    i2p = jnp.concatenate([idx2, pad]).reshape(1, _EP)
    return k(table_a, table_b, i1p, i2p)


def _edge_body(ga_ref, gb_ref, ef_ref, w1c_ref, w2_ref, w3_ref, b23_ref,
               ek_ref, stats_ref, acc_ref):
    i = pl.program_id(0)

    @pl.when(i == 0)
    def _():
        acc_ref[...] = jnp.zeros((8, _DIM), _F32)

    h = ga_ref[...] + gb_ref[...] + jnp.dot(ef_ref[...], w1c_ref[...], **_PF32)
    h = _leaky(h)
    h = jnp.dot(h, w2_ref[...], **_PF32) + b23_ref[0:1, :]
    h = _leaky(h)
    ek = jnp.dot(h, w3_ref[...], **_PF32) + b23_ref[1:2, :]
    ek_ref[...] = ek

    psum = jnp.sum(ek, axis=0, keepdims=True)
    psq = jnp.sum(ek * ek, axis=0, keepdims=True)
    acc_ref[...] += jnp.concatenate(
        [psum, psq, jnp.zeros((6, _DIM), _F32)], axis=0)

    @pl.when(i == pl.num_programs(0) - 1)
    def _():
        stats_ref[...] = acc_ref[...]


def _edge_pass(ga, gb, ef, w1c, w2, w3, b23):
    nblk = _E // _BE
    blk = lambda i: (i, 0)
    full = lambda i: (0, 0)
    return pl.pallas_call(
        _edge_body,
        grid=(nblk,),
        in_specs=[
            pl.BlockSpec((_BE, _DIM), blk),
            pl.BlockSpec((_BE, _DIM), blk),
            pl.BlockSpec((_BE, _DIM), blk),
            pl.BlockSpec((_DIM, _DIM), full),
            pl.BlockSpec((_DIM, _DIM), full),
            pl.BlockSpec((_DIM, _DIM), full),
            pl.BlockSpec((8, _DIM), full),
        ],
        out_specs=(
            pl.BlockSpec((_BE, _DIM), blk),
            pl.BlockSpec((8, _DIM), full),
        ),
        out_shape=(
            jax.ShapeDtypeStruct((_E, _DIM), _F32),
            jax.ShapeDtypeStruct((8, _DIM), _F32),
        ),
        scratch_shapes=[pltpu.VMEM((8, _DIM), _F32)],
        compiler_params=pltpu.CompilerParams(
            dimension_semantics=("arbitrary",)),
    )(ga, gb, ef, w1c, w2, w3, b23)


def _sc_scatter(ek, idx1):
    mesh = plsc.VectorSubcoreMesh(core_axis_name="c", subcore_axis_name="s")
    cp = pltpu.CompilerParams()
    if "needs_layout_passes" in pltpu.CompilerParams.__dataclass_fields__:
        cp = dataclasses.replace(cp, needs_layout_passes=False)

    @functools.partial(
        pl.kernel,
        compiler_params=cp,
        out_type=(
            jax.ShapeDtypeStruct((_NC, _NT, _DIM), _F32),
            jax.ShapeDtypeStruct((_NW, _NT // _DIM, _DIM), _F32),
        ),
        mesh=mesh,
        scratch_types=[
            pltpu.VMEM((_CH,), jnp.int32),
            pltpu.VMEM((_CH, _DIM), _F32),
            pltpu.VMEM((_NT // _DIM, _DIM), _F32),
            pltpu.VMEM_SHARED((_NT, _DIM), _F32),
        ],
    )
    def k(ek_hbm, idx_hbm, osum_hbm, ocnt_hbm, idx_v, ek_v, hist_v, ssum):
        c = lax.axis_index("c")
        s = lax.axis_index("s")
        z16 = jnp.zeros((16,), _F32)

        @pl.loop(0, _NT // _DIM)
        def _(r):
            @pl.loop(0, _DIM, step=16)
            def _(cc):
                hist_v[r, pl.ds(cc, 16)] = z16

        @pl.loop(0, _CH)
        def _(r):
            @pl.loop(0, _DIM, step=16)
            def _(cc):
                ek_v[r, pl.ds(cc, 16)] = z16

        @pl.loop(0, _STR, step=_CH)
        def _(kk):
            pltpu.sync_copy(ek_v, ssum.at[pl.ds(s * _STR + kk, _CH)])

        plsc.subcore_barrier()

        wid = c * _NS + s

        def do_chunk(off):
            pltpu.sync_copy(idx_hbm.at[pl.ds(off, _CH)], idx_v)
            pltpu.sync_copy(ek_hbm.at[pl.ds(off, _CH)], ek_v)
            pltpu.sync_copy(ek_v, ssum.at[idx_v], add=True)
            for i in range(_CH // 16):
                v = idx_v[pl.ds(i * 16, 16)]
                cntv, lastm = plsc.scan_count(v)
                plsc.addupdate_scatter(
                    hist_v,
                    [lax.shift_right_logical(v, 7), lax.bitwise_and(v, 127)],
                    cntv.astype(_F32), mask=lastm)

        base = wid * _NCH * _CH

        @pl.loop(0, _NCH)
        def _(j):
            do_chunk(base + j * _CH)

        @pl.when(wid < (_E // _CH) - _NW * _NCH)
        def _():
            do_chunk((_NW * _NCH + wid) * _CH)

        plsc.subcore_barrier()

        pltpu.sync_copy(hist_v, ocnt_hbm.at[wid])

        @pl.loop(0, _STR, step=_CH)
        def _(kk):
            row = s * _STR + kk
            pltpu.sync_copy(ssum.at[pl.ds(row, _CH)], ek_v)
            pltpu.sync_copy(ek_v, osum_hbm.at[c, pl.ds(row, _CH)])

    return k(ek, idx1)


def _edge2_body(ef_ref, ek_ref, stats_ref, gb1_ref, out_ref):
    m1 = stats_ref[0:1, :] / _E
    v1 = stats_ref[1:2, :] / _E - m1 * m1
    a1 = gb1_ref[0:1, :] * lax.rsqrt(v1 + _EPS)
    c1 = gb1_ref[1:2, :] - a1 * m1
    out_ref[...] = ef_ref[...] + a1 * ek_ref[...] + c1


def _edge2_pass(ef, ek, stats, gb1):
    nblk = _E // _BE
    blk = lambda i: (i, 0)
    full = lambda i: (0, 0)
    return pl.pallas_call(
        _edge2_body,
        grid=(nblk,),
        in_specs=[
            pl.BlockSpec((_BE, _DIM), blk),
            pl.BlockSpec((_BE, _DIM), blk),
            pl.BlockSpec((8, _DIM), full),
            pl.BlockSpec((8, _DIM), full),
        ],
        out_specs=pl.BlockSpec((_BE, _DIM), blk),
        out_shape=jax.ShapeDtypeStruct((_E, _DIM), _F32),
        compiler_params=pltpu.CompilerParams(
            dimension_semantics=("arbitrary",)),
    )(ef, ek, stats, gb1)


def _node_body(ps_ref, pc_ref, node_ref, stats_ref, m1h_ref, m2h_ref,
               wp_ref, wq_ref, w2_ref, w3_ref, misc_ref, out_ref):
    m1 = stats_ref[0:1, :] / _E
    v1 = stats_ref[1:2, :] / _E - m1 * m1
    a1 = misc_ref[3:4, :] * lax.rsqrt(v1 + _EPS)
    c1 = misc_ref[4:5, :] - a1 * m1

    seg = ps_ref[0, 0:_N, :] + ps_ref[1, 0:_N, :]
    hist = jnp.sum(pc_ref[...], axis=0)
    cnt = jnp.sum(jnp.dot(m1h_ref[...], hist, **_PF32) * m2h_ref[...],
                  axis=1, keepdims=True)
    vbar = (a1 * seg + c1 * cnt) / jnp.maximum(cnt, 1.0)

    node = node_ref[...]
    h = (jnp.dot(vbar, wp_ref[...], **_PF32)
         + jnp.dot(node, wq_ref[...], **_PF32) + misc_ref[0:1, :])
    h = _leaky(h)
    h = jnp.dot(h, w2_ref[...], **_PF32) + misc_ref[1:2, :]
    h = _leaky(h)
    vi = jnp.dot(h, w3_ref[...], **_PF32) + misc_ref[2:3, :]

    m2 = jnp.mean(vi, axis=0, keepdims=True)
    v2 = jnp.mean(vi * vi, axis=0, keepdims=True) - m2 * m2
    out_ref[...] = (node + misc_ref[5:6, :] * (vi - m2) * lax.rsqrt(v2 + _EPS)
                    + misc_ref[6:7, :])


def _node_pass(psum, pcnt, node_fea, stats, m1h, m2h, wp, wq, w2, w3, misc):
    return pl.pallas_call(
        _node_body,
        out_shape=jax.ShapeDtypeStruct((_N, _DIM), _F32),
    )(psum, pcnt, node_fea, stats, m1h, m2h, wp, wq, w2, w3, misc)


def kernel(node_fea, idx1, idx2, edge_fea, params):
    p = params
    w1t = p['We1'].T
    w1a, w1b, w1c = w1t[:_DIM], w1t[_DIM:2 * _DIM], w1t[2 * _DIM:]
    w2t, w3t = p['We2'].T, p['We3'].T
    wv1t = p['Wv1'].T
    wp, wq = wv1t[:_DIM], wv1t[_DIM:]
    wv2t, wv3t = p['Wv2'].T, p['Wv3'].T

    def row8(*rows):
        out = jnp.zeros((8, _DIM), _F32)
        for i, r in enumerate(rows):
            out = out.at[i].set(r)
        return out

    be1 = row8(p['be1'])
    b23 = row8(p['be2'], p['be3'])
    gb1 = row8(p['g1'], p['beta1'])
    misc = row8(p['bv1'], p['bv2'], p['bv3'], p['g1'], p['beta1'],
                p['g2'], p['beta2'])

    ta, tb = _pretransform(node_fea, w1a, w1b, be1)
    ga, gb = _sc_gather(ta, tb, idx1, idx2)
    ek, stats = _edge_pass(ga, gb, edge_fea, w1c, w2t, w3t, b23)
    psum, pcnt = _sc_scatter(ek, idx1)
    edge_new = _edge2_pass(edge_fea, ek, stats, gb1)

    n_ids = jnp.arange(_N, dtype=jnp.int32)
    m1h = (n_ids[:, None] // _DIM == jnp.arange(_NT // _DIM)[None, :]
           ).astype(_F32)
    m2h = (n_ids[:, None] % _DIM == jnp.arange(_DIM)[None, :]).astype(_F32)

    node_new = _node_pass(psum, pcnt, node_fea, stats, m1h, m2h,
                          wp, wq, wv2t, wv3t, misc)
    return (node_new, edge_new)


# speedup vs baseline: 4.7611x; 1.1914x over previous
import dataclasses
import functools

import jax
import jax.numpy as jnp
from jax import lax
from jax.experimental import pallas as pl
from jax.experimental.pallas import tpu as pltpu
from jax.experimental.pallas import tpu_sc as plsc

_DIM = 128
_N = 10000
_E = 320000
_EPS = 1e-5

_F32 = jnp.float32
_PF32 = dict(preferred_element_type=jnp.float32)

_NC, _NS = 2, 16
_NW = _NC * _NS
_GW = 128
_EP = 327680
_CH = 128
_NCH = _E // _CH // _NW
_NT = 10240
_STR = _NT // _NS

_BE = 4000


def _leaky(x):
    return jnp.where(x >= 0, x, 0.2 * x)


def _pre_body(node_ref, w1a_ref, w1b_ref, be1_ref, a_ref, b_ref):
    n = node_ref[...]
    a_ref[...] = jnp.dot(n, w1a_ref[...], **_PF32) + be1_ref[0:1, :]
    b_ref[...] = jnp.dot(n, w1b_ref[...], **_PF32)


def _pretransform(node_fea, w1a, w1b, be1):
    return pl.pallas_call(
        _pre_body,
        out_shape=(
            jax.ShapeDtypeStruct((_N, _DIM), _F32),
            jax.ShapeDtypeStruct((_N, _DIM), _F32),
        ),
    )(node_fea, w1a, w1b, be1)


def _sc_gather(table_a, table_b, idx1, idx2):
    mesh = plsc.VectorSubcoreMesh(core_axis_name="c", subcore_axis_name="s")

    @functools.partial(
        pl.kernel,
        out_type=(
            jax.ShapeDtypeStruct((_EP, _DIM), _F32),
            jax.ShapeDtypeStruct((_EP, _DIM), _F32),
        ),
        mesh=mesh,
        scratch_types=[pltpu.SemaphoreType.DMA, pltpu.SemaphoreType.DMA],
    )
    def k(ta_hbm, tb_hbm, i1_hbm, i2_hbm, oa_hbm, ob_hbm, sema, semb):
        def body(i1_vmem, i2_vmem, oa_vmem, ob_vmem):
            ca = pltpu.async_copy(ta_hbm.at[i1_vmem.at[0]], oa_vmem, sema)
            cb = pltpu.async_copy(tb_hbm.at[i2_vmem.at[0]], ob_vmem, semb)
            ca.wait()
            cb.wait()

        pltpu.emit_pipeline(
            body,
            grid=(_EP // _GW,),
            in_specs=[
                pl.BlockSpec((1, _GW), lambda i: (0, i)),
                pl.BlockSpec((1, _GW), lambda i: (0, i)),
            ],
            out_specs=[
                pl.BlockSpec((_GW, _DIM), lambda i: (i, 0)),
                pl.BlockSpec((_GW, _DIM), lambda i: (i, 0)),
            ],
            core_axis_name=("c", "s"),
            dimension_semantics=(pltpu.PARALLEL,),
        )(i1_hbm, i2_hbm, oa_hbm, ob_hbm)

    pad = (jnp.arange(_EP - _E, dtype=jnp.int32) % _N)
    i1p = jnp.concatenate([idx1, pad]).reshape(1, _EP)
    i2p = jnp.concatenate([idx2, pad]).reshape(1, _EP)
    return k(table_a, table_b, i1p, i2p)


def _edge_body(ga_ref, gb_ref, ef_ref, w1c_ref, w2_ref, w3_ref, b23_ref,
               ek_ref, stats_ref, acc_ref):
    i = pl.program_id(0)

    @pl.when(i == 0)
    def _():
        acc_ref[...] = jnp.zeros((8, _DIM), _F32)

    h = ga_ref[...] + gb_ref[...] + jnp.dot(ef_ref[...], w1c_ref[...], **_PF32)
    h = _leaky(h)
    h = jnp.dot(h, w2_ref[...], **_PF32) + b23_ref[0:1, :]
    h = _leaky(h)
    ek = jnp.dot(h, w3_ref[...], **_PF32) + b23_ref[1:2, :]
    ek_ref[...] = ek

    psum = jnp.sum(ek, axis=0, keepdims=True)
    psq = jnp.sum(ek * ek, axis=0, keepdims=True)
    acc_ref[...] += jnp.concatenate(
        [psum, psq, jnp.zeros((6, _DIM), _F32)], axis=0)

    @pl.when(i == pl.num_programs(0) - 1)
    def _():
        stats_ref[...] = acc_ref[...]


def _edge_pass(ga, gb, ef, w1c, w2, w3, b23):
    nblk = _E // _BE
    blk = lambda i: (i, 0)
    full = lambda i: (0, 0)
    return pl.pallas_call(
        _edge_body,
        grid=(nblk,),
        in_specs=[
            pl.BlockSpec((_BE, _DIM), blk),
            pl.BlockSpec((_BE, _DIM), blk),
            pl.BlockSpec((_BE, _DIM), blk),
            pl.BlockSpec((_DIM, _DIM), full),
            pl.BlockSpec((_DIM, _DIM), full),
            pl.BlockSpec((_DIM, _DIM), full),
            pl.BlockSpec((8, _DIM), full),
        ],
        out_specs=(
            pl.BlockSpec((_BE, _DIM), blk),
            pl.BlockSpec((8, _DIM), full),
        ),
        out_shape=(
            jax.ShapeDtypeStruct((_E, _DIM), _F32),
            jax.ShapeDtypeStruct((8, _DIM), _F32),
        ),
        scratch_shapes=[pltpu.VMEM((8, _DIM), _F32)],
        compiler_params=pltpu.CompilerParams(
            dimension_semantics=("arbitrary",)),
    )(ga, gb, ef, w1c, w2, w3, b23)


def _sc_scatter(ek, idx1):
    mesh = plsc.VectorSubcoreMesh(core_axis_name="c", subcore_axis_name="s")
    cp = pltpu.CompilerParams()
    if "needs_layout_passes" in pltpu.CompilerParams.__dataclass_fields__:
        cp = dataclasses.replace(cp, needs_layout_passes=False)

    @functools.partial(
        pl.kernel,
        compiler_params=cp,
        out_type=(
            jax.ShapeDtypeStruct((_NC, _NT, _DIM), _F32),
            jax.ShapeDtypeStruct((_NW, _NT // _DIM, _DIM), _F32),
        ),
        mesh=mesh,
        scratch_types=[
            pltpu.VMEM((_CH,), jnp.int32),
            pltpu.VMEM((_CH,), jnp.int32),
            pltpu.VMEM((_CH, _DIM), _F32),
            pltpu.VMEM((_CH, _DIM), _F32),
            pltpu.VMEM((_NT // _DIM, _DIM), _F32),
            pltpu.VMEM_SHARED((_NT, _DIM), _F32),
            pltpu.SemaphoreType.DMA,
            pltpu.SemaphoreType.DMA,
            pltpu.SemaphoreType.DMA,
            pltpu.SemaphoreType.DMA,
        ],
    )
    def k(ek_hbm, idx_hbm, osum_hbm, ocnt_hbm, idx_v, idx2_v, ek_v, ek2_v,
          hist_v, ssum, lsem0, lsem1, ssem0, ssem1):
        lsem = (lsem0, lsem1)
        ssem = (ssem0, ssem1)
        c = lax.axis_index("c")
        s = lax.axis_index("s")
        z16 = jnp.zeros((16,), _F32)

        @pl.loop(0, _NT // _DIM)
        def _(r):
            @pl.loop(0, _DIM, step=16)
            def _(cc):
                hist_v[r, pl.ds(cc, 16)] = z16

        @pl.loop(0, _CH)
        def _(r):
            @pl.loop(0, _DIM, step=16)
            def _(cc):
                ek_v[r, pl.ds(cc, 16)] = z16

        @pl.loop(0, _STR, step=_CH)
        def _(kk):
            pltpu.sync_copy(ek_v, ssum.at[pl.ds(s * _STR + kk, _CH)])

        plsc.subcore_barrier()

        wid = c * _NS + s

        def hist_update(idx_ref):
            for i in range(_CH // 16):
                v = idx_ref[pl.ds(i * 16, 16)]
                cntv, lastm = plsc.scan_count(v)
                plsc.addupdate_scatter(
                    hist_v,
                    [lax.shift_right_logical(v, 7), lax.bitwise_and(v, 127)],
                    cntv.astype(_F32), mask=lastm)

        idxb = (idx_v, idx2_v)
        ekb = (ek_v, ek2_v)

        def start_load(bi, off):
            pltpu.async_copy(idx_hbm.at[pl.ds(off, _CH)], idxb[bi], lsem[bi])
            pltpu.async_copy(ek_hbm.at[pl.ds(off, _CH)], ekb[bi], lsem[bi])

        def wait_load(bi):
            pltpu.make_async_copy(
                idx_hbm.at[pl.ds(0, _CH)], idxb[bi], lsem[bi]).wait()
            pltpu.make_async_copy(
                ek_hbm.at[pl.ds(0, _CH)], ekb[bi], lsem[bi]).wait()

        def start_scatter(bi):
            pltpu.async_copy(ekb[bi], ssum.at[idxb[bi]], ssem[bi], add=True)

        def wait_scatter(bi):
            pltpu.make_async_copy(ekb[bi], ssum.at[idxb[bi]], ssem[bi]).wait()

        base = wid * _NCH * _CH
        start_load(0, base)
        start_load(1, base + _CH)

        @pl.loop(0, _NCH // 2)
        def _(jj):
            off = base + jj * 2 * _CH
            wait_load(0)
            hist_update(idxb[0])
            start_scatter(0)
            wait_load(1)
            hist_update(idxb[1])
            start_scatter(1)

            @pl.when(jj < _NCH // 2 - 1)
            def _():
                wait_scatter(0)
                start_load(0, off + 2 * _CH)
                wait_scatter(1)
                start_load(1, off + 3 * _CH)

        wait_scatter(0)
        wait_scatter(1)

        @pl.when(wid < (_E // _CH) - _NW * _NCH)
        def _():
            off = (_NW * _NCH + wid) * _CH
            pltpu.sync_copy(idx_hbm.at[pl.ds(off, _CH)], idx_v)
            pltpu.sync_copy(ek_hbm.at[pl.ds(off, _CH)], ek_v)
            pltpu.sync_copy(ek_v, ssum.at[idx_v], add=True)
            hist_update(idx_v)

        plsc.subcore_barrier()

        pltpu.sync_copy(hist_v, ocnt_hbm.at[wid])

        @pl.loop(0, _STR, step=_CH)
        def _(kk):
            row = s * _STR + kk
            pltpu.sync_copy(ssum.at[pl.ds(row, _CH)], ek_v)
            pltpu.sync_copy(ek_v, osum_hbm.at[c, pl.ds(row, _CH)])

    return k(ek, idx1)


def _edge2_body(ef_ref, ek_ref, stats_ref, gb1_ref, out_ref):
    m1 = stats_ref[0:1, :] / _E
    v1 = stats_ref[1:2, :] / _E - m1 * m1
    a1 = gb1_ref[0:1, :] * lax.rsqrt(v1 + _EPS)
    c1 = gb1_ref[1:2, :] - a1 * m1
    out_ref[...] = ef_ref[...] + a1 * ek_ref[...] + c1


def _edge2_pass(ef, ek, stats, gb1):
    nblk = _E // _BE
    blk = lambda i: (i, 0)
    full = lambda i: (0, 0)
    return pl.pallas_call(
        _edge2_body,
        grid=(nblk,),
        in_specs=[
            pl.BlockSpec((_BE, _DIM), blk),
            pl.BlockSpec((_BE, _DIM), blk),
            pl.BlockSpec((8, _DIM), full),
            pl.BlockSpec((8, _DIM), full),
        ],
        out_specs=pl.BlockSpec((_BE, _DIM), blk),
        out_shape=jax.ShapeDtypeStruct((_E, _DIM), _F32),
        compiler_params=pltpu.CompilerParams(
            dimension_semantics=("arbitrary",)),
    )(ef, ek, stats, gb1)


def _node_body(ps_ref, pc_ref, node_ref, stats_ref, m1h_ref, m2h_ref,
               wp_ref, wq_ref, w2_ref, w3_ref, misc_ref, out_ref):
    m1 = stats_ref[0:1, :] / _E
    v1 = stats_ref[1:2, :] / _E - m1 * m1
    a1 = misc_ref[3:4, :] * lax.rsqrt(v1 + _EPS)
    c1 = misc_ref[4:5, :] - a1 * m1

    seg = ps_ref[0, 0:_N, :] + ps_ref[1, 0:_N, :]
    hist = jnp.sum(pc_ref[...], axis=0)
    cnt = jnp.sum(jnp.dot(m1h_ref[...], hist, **_PF32) * m2h_ref[...],
                  axis=1, keepdims=True)
    vbar = (a1 * seg + c1 * cnt) / jnp.maximum(cnt, 1.0)

    node = node_ref[...]
    h = (jnp.dot(vbar, wp_ref[...], **_PF32)
         + jnp.dot(node, wq_ref[...], **_PF32) + misc_ref[0:1, :])
    h = _leaky(h)
    h = jnp.dot(h, w2_ref[...], **_PF32) + misc_ref[1:2, :]
    h = _leaky(h)
    vi = jnp.dot(h, w3_ref[...], **_PF32) + misc_ref[2:3, :]

    m2 = jnp.mean(vi, axis=0, keepdims=True)
    v2 = jnp.mean(vi * vi, axis=0, keepdims=True) - m2 * m2
    out_ref[...] = (node + misc_ref[5:6, :] * (vi - m2) * lax.rsqrt(v2 + _EPS)
                    + misc_ref[6:7, :])


def _node_pass(psum, pcnt, node_fea, stats, m1h, m2h, wp, wq, w2, w3, misc):
    return pl.pallas_call(
        _node_body,
        out_shape=jax.ShapeDtypeStruct((_N, _DIM), _F32),
    )(psum, pcnt, node_fea, stats, m1h, m2h, wp, wq, w2, w3, misc)


def kernel(node_fea, idx1, idx2, edge_fea, params):
    p = params
    w1t = p['We1'].T
    w1a, w1b, w1c = w1t[:_DIM], w1t[_DIM:2 * _DIM], w1t[2 * _DIM:]
    w2t, w3t = p['We2'].T, p['We3'].T
    wv1t = p['Wv1'].T
    wp, wq = wv1t[:_DIM], wv1t[_DIM:]
    wv2t, wv3t = p['Wv2'].T, p['Wv3'].T

    def row8(*rows):
        out = jnp.zeros((8, _DIM), _F32)
        for i, r in enumerate(rows):
            out = out.at[i].set(r)
        return out

    be1 = row8(p['be1'])
    b23 = row8(p['be2'], p['be3'])
    gb1 = row8(p['g1'], p['beta1'])
    misc = row8(p['bv1'], p['bv2'], p['bv3'], p['g1'], p['beta1'],
                p['g2'], p['beta2'])

    ta, tb = _pretransform(node_fea, w1a, w1b, be1)
    ga, gb = _sc_gather(ta, tb, idx1, idx2)
    ek, stats = _edge_pass(ga, gb, edge_fea, w1c, w2t, w3t, b23)
    psum, pcnt = _sc_scatter(ek, idx1)
    edge_new = _edge2_pass(edge_fea, ek, stats, gb1)

    n_ids = jnp.arange(_N, dtype=jnp.int32)
    m1h = (n_ids[:, None] // _DIM == jnp.arange(_NT // _DIM)[None, :]
           ).astype(_F32)
    m2h = (n_ids[:, None] % _DIM == jnp.arange(_DIM)[None, :]).astype(_F32)

    node_new = _node_pass(psum, pcnt, node_fea, stats, m1h, m2h,
                          wp, wq, wv2t, wv3t, misc)
    return (node_new, edge_new)
